# Initial kernel scaffold; baseline (speedup 1.0000x reference)
import jax
import jax.numpy as jnp
from jax.experimental import pallas as pl


def kernel(hidden_states, gate_w, w1, w2, w3):
    raise NotImplementedError("write your pallas kernel here")


# speedup vs baseline: 1.3229x; 1.3229x over previous
import functools

import jax
import jax.numpy as jnp
from jax.experimental import pallas as pl
from jax.experimental.pallas import tpu as pltpu

T = 2048
H = 1024
E = 8
FFN = 4096
K = 2
S = T * K
B_ROW = 256
S_PAD = S + E * B_ROW
NB = S_PAD // B_ROW
F_BLK = 1024
NF = FFN // F_BLK
JITTER = 0.01


def _router_kernel(x_ref, gw_ref, logits_ref, sel1_ref, sel2_ref, m1_ref, m2_ref):
    x = x_ref[...]
    gw = gw_ref[...]
    logits = jax.lax.dot_general(
        x, gw, (((1,), (1,)), ((), ())), preferred_element_type=jnp.float32)
    logits_ref[...] = logits
    iota = jax.lax.broadcasted_iota(jnp.int32, (T, E), 1)
    neg_inf = jnp.float32(-jnp.inf)

    def step(scores):
        m = jnp.max(scores, axis=-1, keepdims=True)
        is_max = scores == m
        sel = jnp.min(jnp.where(is_max, iota, E), axis=-1)
        factor = jnp.maximum(jnp.abs(scores), m)
        mask = ((m - scores) / factor) > (2.0 * JITTER)
        masked = jnp.where(mask, neg_inf, scores)
        mm = jnp.max(masked, axis=-1, keepdims=True)
        ex = jnp.exp(masked - mm)
        probs = ex / jnp.sum(ex, axis=-1, keepdims=True)
        onehot = iota == sel[:, None]
        mult = jnp.sum(jnp.where(onehot, probs, 0.0), axis=-1)
        return mult, sel, onehot

    mult1, sel1, oh1 = step(logits)
    masked2 = jnp.where(oh1, neg_inf, logits)
    mult2, sel2, _ = step(masked2)
    sel1_ref[...] = sel1[:, None]
    sel2_ref[...] = sel2[:, None]
    m1_ref[...] = mult1[:, None]
    m2_ref[...] = mult2[:, None]


def _moe_kernel(ids_ref, be_ref, act_ref, x_ref, w1_ref, w3_ref, w2_ref, g_ref,
                y_ref, xs_ref):
    b = pl.program_id(0)
    f = pl.program_id(1)

    @pl.when(f == 0)
    def _gather():
        def body(i, _):
            t = ids_ref[b * B_ROW + i]
            xs_ref[i, :] = x_ref[t, :]
            return 0
        jax.lax.fori_loop(0, B_ROW, body, 0)

    active = act_ref[b] == 1

    @pl.when(active)
    def _compute():
        xb = xs_ref[...]
        h1 = jax.lax.dot_general(
            xb, w1_ref[0], (((1,), (1,)), ((), ())),
            preferred_element_type=jnp.float32)
        h3 = jax.lax.dot_general(
            xb, w3_ref[0], (((1,), (1,)), ((), ())),
            preferred_element_type=jnp.float32)
        hh = h1 * jax.nn.sigmoid(h1) * h3
        yb = jax.lax.dot_general(
            hh, w2_ref[0], (((1,), (1,)), ((), ())),
            preferred_element_type=jnp.float32)
        yb = yb * g_ref[...]

        @pl.when(f == 0)
        def _init():
            y_ref[...] = yb

        @pl.when(f != 0)
        def _acc():
            y_ref[...] += yb

    @pl.when(jnp.logical_and(jnp.logical_not(active), f == 0))
    def _zero():
        y_ref[...] = jnp.zeros_like(y_ref)


def _combine_kernel(p1_ref, p2_ref, y_ref, o_ref):
    blk = pl.program_id(0)

    def body(i, _):
        t = blk * B_ROW + i
        o_ref[i, :] = y_ref[p1_ref[t], :] + y_ref[p2_ref[t], :]
        return 0
    jax.lax.fori_loop(0, B_ROW, body, 0)


@functools.partial(jax.jit, static_argnames=())
def kernel(hidden_states, gate_w, w1, w2, w3):
    b, s, d = hidden_states.shape
    x = hidden_states.reshape(-1, d)

    logits, sel1, sel2, mult1, mult2 = pl.pallas_call(
        _router_kernel,
        out_shape=[
            jax.ShapeDtypeStruct((T, E), jnp.float32),
            jax.ShapeDtypeStruct((T, 1), jnp.int32),
            jax.ShapeDtypeStruct((T, 1), jnp.int32),
            jax.ShapeDtypeStruct((T, 1), jnp.float32),
            jax.ShapeDtypeStruct((T, 1), jnp.float32),
        ],
    )(x, gate_w)

    e_all = jnp.concatenate([sel1[:, 0], sel2[:, 0]])
    g_all = jnp.concatenate([mult1[:, 0], mult2[:, 0]])
    tok = jnp.concatenate([jnp.arange(T, dtype=jnp.int32)] * 2)
    oh = jax.nn.one_hot(e_all, E, dtype=jnp.int32)
    c_cum = jnp.cumsum(oh, axis=0)
    rank = jnp.take_along_axis(c_cum, e_all[:, None], axis=1)[:, 0] - 1
    counts = c_cum[-1]
    nblk = (counts + B_ROW - 1) // B_ROW
    pstart = jnp.concatenate(
        [jnp.zeros((1,), jnp.int32),
         jnp.cumsum(nblk * B_ROW)[:-1].astype(jnp.int32)])
    dst = (pstart[e_all] + rank).astype(jnp.int32)
    ids = jnp.zeros((S_PAD,), jnp.int32).at[dst].set(tok)
    gates = jnp.zeros((S_PAD,), jnp.float32).at[dst].set(g_all)
    cumb = jnp.cumsum(nblk)
    total = cumb[-1]
    bidx = jnp.minimum(jnp.arange(NB), total - 1)
    be = jnp.searchsorted(cumb, bidx, side='right').astype(jnp.int32)
    act = (jnp.arange(NB) < total).astype(jnp.int32)
    p1 = dst[:T]
    p2 = dst[T:]

    y_sorted = pl.pallas_call(
        _moe_kernel,
        grid_spec=pltpu.PrefetchScalarGridSpec(
            num_scalar_prefetch=3,
            grid=(NB, NF),
            in_specs=[
                pl.BlockSpec((T, H), lambda bb, f, *_: (0, 0)),
                pl.BlockSpec((1, F_BLK, H), lambda bb, f, ids, be, act: (be[bb], f, 0)),
                pl.BlockSpec((1, F_BLK, H), lambda bb, f, ids, be, act: (be[bb], f, 0)),
                pl.BlockSpec((1, H, F_BLK), lambda bb, f, ids, be, act: (be[bb], 0, f)),
                pl.BlockSpec((B_ROW, 1), lambda bb, f, *_: (bb, 0)),
            ],
            out_specs=pl.BlockSpec((B_ROW, H), lambda bb, f, *_: (bb, 0)),
            scratch_shapes=[pltpu.VMEM((B_ROW, H), jnp.float32)],
        ),
        out_shape=jax.ShapeDtypeStruct((S_PAD, H), jnp.float32),
    )(ids, be, act, x, w1, w3, w2, gates[:, None])

    out = pl.pallas_call(
        _combine_kernel,
        grid_spec=pltpu.PrefetchScalarGridSpec(
            num_scalar_prefetch=2,
            grid=(T // B_ROW,),
            in_specs=[pl.BlockSpec((S_PAD, H), lambda bb, p1, p2: (0, 0))],
            out_specs=pl.BlockSpec((B_ROW, H), lambda bb, p1, p2: (bb, 0)),
        ),
        out_shape=jax.ShapeDtypeStruct((T, H), jnp.float32),
    )(p1, p2, y_sorted)

    return out.reshape(b, s, d), logits
